# baseline (speedup 1.0000x reference)
import functools
import math

import jax
import jax.numpy as jnp
from jax import lax
from jax.experimental import pallas as pl
from jax.experimental.pallas import tpu as pltpu
from jax.experimental.pallas import tpu_sc as plsc

_EMB = 128
_SCALE = math.sqrt(_EMB)
_NC = 2
_NS = 16
_NW = _NC * _NS
_LANES = 16


_K = 4


def _emb_body(tok_hbm, table_hbm, out_hbm, idx_v, *scratch):
    rows = list(scratch[:_K])
    s_in = list(scratch[_K:2 * _K])
    s_out = list(scratch[2 * _K:3 * _K])
    wid = lax.axis_index("s") * _NC + lax.axis_index("c")
    seq, nb = idx_v.shape
    b0 = wid * nb

    pltpu.sync_copy(tok_hbm.at[:, pl.ds(b0, nb)], idx_v)

    def scale(r):
        @plsc.parallel_loop(0, nb, unroll=4)
        def _(i):
            for j in range(_EMB // _LANES):
                sl = pl.ds(j * _LANES, _LANES)
                r[i, sl] = r[i, sl] * _SCALE

    def gather(h, b):
        pltpu.async_copy(table_hbm.at[idx_v.at[h]], rows[b], s_in[b])

    def slot(g, b, drain, issue):
        b2 = (b + 2) % _K
        if drain:
            pltpu.make_async_copy(
                rows[b2], out_hbm.at[g, pl.ds(b0, nb)], s_out[b2]
            ).wait()
        if issue:
            gather(g + 2, b2)
        pltpu.make_async_copy(table_hbm.at[idx_v.at[g]], rows[b], s_in[b]).wait()
        scale(rows[b])
        pltpu.async_copy(rows[b], out_hbm.at[g, pl.ds(b0, nb)], s_out[b])

    gather(0, 0)
    gather(1, 1)
    slot(0, 0, False, True)
    slot(1, 1, False, True)
    slot(2, 2, True, True)
    slot(3, 3, True, True)

    def outer(u, carry):
        g = 4 * u
        for b in range(_K):
            slot(g + b, b, True, True)
        return carry

    lax.fori_loop(1, (seq - 2) // 4, outer, 0)

    slot(seq - 2, (seq - 2) % _K, True, False)
    slot(seq - 1, (seq - 1) % _K, True, False)
    for g in (seq - 2, seq - 1):
        b = g % _K
        pltpu.make_async_copy(rows[b], out_hbm.at[g, pl.ds(b0, nb)], s_out[b]).wait()


def kernel(tokens, table):
    b, l = tokens.shape
    assert b % _NW == 0 and l >= 6 and (l - 2) % 4 == 0
    nb = b // _NW
    tok_t = tokens.T.astype(jnp.int32)

    grid_kernel = functools.partial(
        pl.kernel,
        mesh=plsc.VectorSubcoreMesh(core_axis_name="c", subcore_axis_name="s"),
        out_type=jax.ShapeDtypeStruct((l, b, _EMB), jnp.float32),
        scratch_types=(
            [pltpu.VMEM((l, nb), jnp.int32)]
            + [pltpu.VMEM((nb, _EMB), jnp.float32) for _ in range(_K)]
            + [pltpu.SemaphoreType.DMA for _ in range(2 * _K)]
        ),
    )(_emb_body)

    out = grid_kernel(tok_t, table)
    return jnp.transpose(out, (1, 0, 2))


# speedup vs baseline: 1.0018x; 1.0018x over previous
import functools
import math

import jax
import jax.numpy as jnp
from jax import lax
from jax.experimental import pallas as pl
from jax.experimental.pallas import tpu as pltpu
from jax.experimental.pallas import tpu_sc as plsc

_EMB = 128
_SCALE = math.sqrt(_EMB)
_NC = 2
_NS = 16
_NW = _NC * _NS
_LANES = 16


_K = 4


def _emb_body(tok_hbm, table_hbm, out_hbm, idx_v, *scratch):
    rows = list(scratch[:_K])
    s_in = list(scratch[_K:2 * _K])
    s_out = list(scratch[2 * _K:3 * _K])
    wid = lax.axis_index("s") * _NC + lax.axis_index("c")
    seq, nb = idx_v.shape
    b0 = wid * nb

    pltpu.sync_copy(tok_hbm.at[:, pl.ds(b0, nb)], idx_v)

    def scale(r):
        def rows4(i, c):
            for k in range(4):
                for j in range(_EMB // _LANES):
                    sl = pl.ds(j * _LANES, _LANES)
                    r[4 * i + k, sl] = r[4 * i + k, sl] * _SCALE
            return c

        lax.fori_loop(0, nb // 4, rows4, 0)

    def gather(h, b):
        pltpu.async_copy(table_hbm.at[idx_v.at[h]], rows[b], s_in[b])

    def slot(g, b, drain, issue):
        b2 = (b + 2) % _K
        if drain:
            pltpu.make_async_copy(
                rows[b2], out_hbm.at[g, pl.ds(b0, nb)], s_out[b2]
            ).wait()
        if issue:
            gather(g + 2, b2)
        pltpu.make_async_copy(table_hbm.at[idx_v.at[g]], rows[b], s_in[b]).wait()
        scale(rows[b])
        pltpu.async_copy(rows[b], out_hbm.at[g, pl.ds(b0, nb)], s_out[b])

    gather(0, 0)
    gather(1, 1)
    slot(0, 0, False, True)
    slot(1, 1, False, True)
    slot(2, 2, True, True)
    slot(3, 3, True, True)

    def outer(u, carry):
        g = 4 * u
        for b in range(_K):
            slot(g + b, b, True, True)
        return carry

    lax.fori_loop(1, (seq - 2) // 4, outer, 0)

    slot(seq - 2, (seq - 2) % _K, True, False)
    slot(seq - 1, (seq - 1) % _K, True, False)
    for g in (seq - 2, seq - 1):
        b = g % _K
        pltpu.make_async_copy(rows[b], out_hbm.at[g, pl.ds(b0, nb)], s_out[b]).wait()


def kernel(tokens, table):
    b, l = tokens.shape
    assert b % _NW == 0 and l >= 6 and (l - 2) % 4 == 0
    nb = b // _NW
    tok_t = tokens.T.astype(jnp.int32)

    grid_kernel = functools.partial(
        pl.kernel,
        mesh=plsc.VectorSubcoreMesh(core_axis_name="c", subcore_axis_name="s"),
        out_type=jax.ShapeDtypeStruct((l, b, _EMB), jnp.float32),
        scratch_types=(
            [pltpu.VMEM((l, nb), jnp.int32)]
            + [pltpu.VMEM((nb, _EMB), jnp.float32) for _ in range(_K)]
            + [pltpu.SemaphoreType.DMA for _ in range(2 * _K)]
        ),
    )(_emb_body)

    out = grid_kernel(tok_t, table)
    return jnp.transpose(out, (1, 0, 2))


# speedup vs baseline: 1.0136x; 1.0118x over previous
import functools
import math

import jax
import jax.numpy as jnp
from jax import lax
from jax.experimental import pallas as pl
from jax.experimental.pallas import tpu as pltpu
from jax.experimental.pallas import tpu_sc as plsc

_EMB = 128
_SCALE = math.sqrt(_EMB)
_NC = 2
_NS = 16
_NW = _NC * _NS
_LANES = 16


_K = 6
_LEAD = 3
_LAG = _K - _LEAD


def _emb_body(tok_hbm, table_hbm, out_hbm, idx_v, *scratch):
    rows = list(scratch[:_K])
    s_in = list(scratch[_K:2 * _K])
    s_out = list(scratch[2 * _K:3 * _K])
    wid = lax.axis_index("s") * _NC + lax.axis_index("c")
    seq, nb = idx_v.shape
    b0 = wid * nb

    pltpu.sync_copy(tok_hbm.at[:, pl.ds(b0, nb)], idx_v)

    def scale(r):
        def row(i, c):
            for j in range(_EMB // _LANES):
                sl = pl.ds(j * _LANES, _LANES)
                r[i, sl] = r[i, sl] * _SCALE
            return c

        lax.fori_loop(0, nb, row, 0)

    def gather(h, b):
        pltpu.async_copy(table_hbm.at[idx_v.at[h]], rows[b], s_in[b])

    def slot(g, b, drain, issue):
        b2 = (b + _LEAD) % _K
        if drain:
            pltpu.make_async_copy(
                rows[b2], out_hbm.at[g, pl.ds(b0, nb)], s_out[b2]
            ).wait()
        if issue:
            gather(g + _LEAD, b2)
        pltpu.make_async_copy(table_hbm.at[idx_v.at[g]], rows[b], s_in[b]).wait()
        scale(rows[b])
        pltpu.async_copy(rows[b], out_hbm.at[g, pl.ds(b0, nb)], s_out[b])

    for g in range(_LEAD):
        gather(g, g)
    for g in range(_K):
        slot(g, g, g >= _LAG, True)

    def outer(u, carry):
        g = _K * u
        for b in range(_K):
            slot(g + b, b, True, True)
        return carry

    hi = ((seq - _LEAD) // _K) * _K
    lax.fori_loop(1, hi // _K, outer, 0)

    for g in range(hi, seq):
        slot(g, g % _K, True, g + _LEAD < seq)
    for g in range(seq - _LAG, seq):
        b = g % _K
        pltpu.make_async_copy(rows[b], out_hbm.at[g, pl.ds(b0, nb)], s_out[b]).wait()


def kernel(tokens, table):
    b, l = tokens.shape
    assert b % _NW == 0 and l >= 2 * _K
    nb = b // _NW
    tok_t = tokens.T.astype(jnp.int32)

    grid_kernel = functools.partial(
        pl.kernel,
        mesh=plsc.VectorSubcoreMesh(core_axis_name="c", subcore_axis_name="s"),
        out_type=jax.ShapeDtypeStruct((l, b, _EMB), jnp.float32),
        scratch_types=(
            [pltpu.VMEM((l, nb), jnp.int32)]
            + [pltpu.VMEM((nb, _EMB), jnp.float32) for _ in range(_K)]
            + [pltpu.SemaphoreType.DMA for _ in range(2 * _K)]
        ),
    )(_emb_body)

    out = grid_kernel(tok_t, table)
    return jnp.transpose(out, (1, 0, 2))


# speedup vs baseline: 1.0169x; 1.0032x over previous
import functools
import math

import jax
import jax.numpy as jnp
from jax import lax
from jax.experimental import pallas as pl
from jax.experimental.pallas import tpu as pltpu
from jax.experimental.pallas import tpu_sc as plsc

_EMB = 128
_SCALE = math.sqrt(_EMB)
_NC = 2
_NS = 16
_NW = _NC * _NS
_LANES = 16


_K = 4
_LEAD = 2
_LAG = _K - _LEAD


def _emb_body(tok_hbm, table_hbm, out_hbm, idx_v, *scratch):
    rows = list(scratch[:_K])
    s_in = list(scratch[_K:2 * _K])
    s_out = list(scratch[2 * _K:3 * _K])
    wid = lax.axis_index("s") * _NC + lax.axis_index("c")
    seq, nb = idx_v.shape
    b0 = wid * nb

    pltpu.sync_copy(tok_hbm.at[:, pl.ds(b0, nb)], idx_v)

    def scale(r):
        def row(i, c):
            for j in range(_EMB // _LANES):
                sl = pl.ds(j * _LANES, _LANES)
                r[i, sl] = r[i, sl] * _SCALE
            return c

        lax.fori_loop(0, nb, row, 0)

    def gather(h, b):
        pltpu.async_copy(table_hbm.at[idx_v.at[h]], rows[b], s_in[b])

    def slot(g, b, drain, issue):
        b2 = (b + _LEAD) % _K
        if drain:
            pltpu.make_async_copy(
                rows[b2], out_hbm.at[g, pl.ds(b0, nb)], s_out[b2]
            ).wait()
        if issue:
            gather(g + _LEAD, b2)
        pltpu.make_async_copy(table_hbm.at[idx_v.at[g]], rows[b], s_in[b]).wait()
        scale(rows[b])
        pltpu.async_copy(rows[b], out_hbm.at[g, pl.ds(b0, nb)], s_out[b])

    for g in range(_LEAD):
        gather(g, g)
    for g in range(_K):
        slot(g, g, g >= _LAG, True)

    def outer(u, carry):
        g = _K * u
        for b in range(_K):
            slot(g + b, b, True, True)
        return carry

    hi = ((seq - _LEAD) // _K) * _K
    lax.fori_loop(1, hi // _K, outer, 0)

    for g in range(hi, seq):
        slot(g, g % _K, True, g + _LEAD < seq)
    for g in range(seq - _LAG, seq):
        b = g % _K
        pltpu.make_async_copy(rows[b], out_hbm.at[g, pl.ds(b0, nb)], s_out[b]).wait()


def kernel(tokens, table):
    b, l = tokens.shape
    assert b % _NW == 0 and l >= 2 * _K
    nb = b // _NW
    tok_t = tokens.T.astype(jnp.int32)

    grid_kernel = functools.partial(
        pl.kernel,
        mesh=plsc.VectorSubcoreMesh(core_axis_name="c", subcore_axis_name="s"),
        out_type=jax.ShapeDtypeStruct((l, b, _EMB), jnp.float32),
        scratch_types=(
            [pltpu.VMEM((l, nb), jnp.int32)]
            + [pltpu.VMEM((nb, _EMB), jnp.float32) for _ in range(_K)]
            + [pltpu.SemaphoreType.DMA for _ in range(2 * _K)]
        ),
    )(_emb_body)

    out = grid_kernel(tok_t, table)
    return jnp.transpose(out, (1, 0, 2))
